# initial kernel scaffold (unmeasured)
import jax
import jax.numpy as jnp
from jax import lax
from jax.experimental import pallas as pl
from jax.experimental.pallas import tpu as pltpu

N_DEV = 4


def _local_matmul(A, B):
    M, K = A.shape
    _, N = B.shape
    BM = 512

    def mm_body(a_ref, b_ref, p_ref):
        p_ref[...] = jnp.dot(
            a_ref[...], b_ref[...], preferred_element_type=jnp.float32
        ).astype(jnp.bfloat16)

    return pl.pallas_call(
        mm_body,
        grid=(M // BM,),
        in_specs=[
            pl.BlockSpec((BM, K), lambda i: (i, 0)),
            pl.BlockSpec((K, N), lambda i: (0, 0)),
        ],
        out_specs=pl.BlockSpec((BM, N), lambda i: (i, 0)),
        out_shape=jax.ShapeDtypeStruct((M, N), jnp.bfloat16),
    )(A.astype(jnp.bfloat16), B.astype(jnp.bfloat16))


def _ring_allreduce(P):
    M, N = P.shape
    CM = M // N_DEV

    def body(p_ref, o_ref, comm_ref, pch_ref, send_sems, recv_sems, local_sem):
        me = lax.axis_index("i")
        left = (me + N_DEV - 1) % N_DEV
        right = (me + 1) % N_DEV

        barrier_sem = pltpu.get_barrier_semaphore()
        for nbr in (left, right):
            pl.semaphore_signal(
                barrier_sem, inc=1,
                device_id=(nbr,), device_id_type=pl.DeviceIdType.MESH,
            )
        pl.semaphore_wait(barrier_sem, 2)

        def load_chunk(c, dst):
            cp = pltpu.make_async_copy(
                p_ref.at[pl.ds(c * CM, CM), :], dst, local_sem
            )
            cp.start()
            cp.wait()

        def acc_into(slot, c):
            load_chunk(c, pch_ref)
            comm_ref[slot] = (
                comm_ref[slot].astype(jnp.float32)
                + pch_ref[...].astype(jnp.float32)
            ).astype(jnp.bfloat16)

        load_chunk(me, comm_ref.at[0])

        for s in range(2 * (N_DEV - 1)):
            send_slot = s % 2
            recv_slot = (s + 1) % 2
            rdma = pltpu.make_async_remote_copy(
                src_ref=comm_ref.at[send_slot],
                dst_ref=comm_ref.at[recv_slot],
                send_sem=send_sems.at[s],
                recv_sem=recv_sems.at[s],
                device_id=(right,),
                device_id_type=pl.DeviceIdType.MESH,
            )
            rdma.start()
            rdma.wait()

            if s < N_DEV - 2:
                acc_into(recv_slot, (me - s - 1) % N_DEV)
            elif s == N_DEV - 2:
                c = (me + 1) % N_DEV
                acc_into(recv_slot, c)
                o_ref[pl.ds(c * CM, CM), :] = comm_ref[recv_slot]
            else:
                c = (me - (s - (N_DEV - 1))) % N_DEV
                o_ref[pl.ds(c * CM, CM), :] = comm_ref[recv_slot]

    return pl.pallas_call(
        body,
        out_shape=jax.ShapeDtypeStruct((M, N), jnp.bfloat16),
        in_specs=[pl.BlockSpec(memory_space=pltpu.ANY)],
        out_specs=pl.BlockSpec(memory_space=pltpu.VMEM),
        scratch_shapes=[
            pltpu.VMEM((2, CM, N), jnp.bfloat16),
            pltpu.VMEM((CM, N), jnp.bfloat16),
            pltpu.SemaphoreType.DMA((2 * (N_DEV - 1),)),
            pltpu.SemaphoreType.DMA((2 * (N_DEV - 1),)),
            pltpu.SemaphoreType.DMA,
        ],
        compiler_params=pltpu.CompilerParams(collective_id=0),
    )(P)


def kernel(A, B):
    P = _local_matmul(A, B)
    return _ring_allreduce(P)


# baseline (device time: 721146 ns/iter reference)
import jax
import jax.numpy as jnp
from jax import lax
from jax.experimental import pallas as pl
from jax.experimental.pallas import tpu as pltpu

N_DEV = 4


def _local_matmul(A, B):
    M, K = A.shape
    _, N = B.shape
    BM = 512

    def mm_body(a_ref, b_ref, p_ref):
        p_ref[...] = jnp.dot(
            a_ref[...], b_ref[...], preferred_element_type=jnp.float32
        ).astype(jnp.bfloat16)

    return pl.pallas_call(
        mm_body,
        grid=(M // BM,),
        in_specs=[
            pl.BlockSpec((BM, K), lambda i: (i, 0)),
            pl.BlockSpec((K, N), lambda i: (0, 0)),
        ],
        out_specs=pl.BlockSpec((BM, N), lambda i: (i, 0)),
        out_shape=jax.ShapeDtypeStruct((M, N), jnp.bfloat16),
        compiler_params=pltpu.CompilerParams(
            vmem_limit_bytes=64 * 1024 * 1024
        ),
    )(A.astype(jnp.bfloat16), B.astype(jnp.bfloat16))


def _ring_allreduce(P):
    M, N = P.shape
    CM = M // N_DEV

    def body(p_ref, o_ref, comm_ref, pch_ref, send_sems, recv_sems, local_sem):
        me = lax.axis_index("i")
        left = (me + N_DEV - 1) % N_DEV
        right = (me + 1) % N_DEV

        barrier_sem = pltpu.get_barrier_semaphore()
        for nbr in (left, right):
            pl.semaphore_signal(
                barrier_sem, inc=1,
                device_id=(nbr,), device_id_type=pl.DeviceIdType.MESH,
            )
        pl.semaphore_wait(barrier_sem, 2)

        def load_chunk(c, dst):
            cp = pltpu.make_async_copy(
                p_ref.at[pl.ds(c * CM, CM), :], dst, local_sem
            )
            cp.start()
            cp.wait()

        def acc_into(slot, c):
            load_chunk(c, pch_ref)
            comm_ref[slot] = (
                comm_ref[slot].astype(jnp.float32)
                + pch_ref[...].astype(jnp.float32)
            ).astype(jnp.bfloat16)

        load_chunk(me, comm_ref.at[0])

        for s in range(2 * (N_DEV - 1)):
            send_slot = s % 2
            recv_slot = (s + 1) % 2
            rdma = pltpu.make_async_remote_copy(
                src_ref=comm_ref.at[send_slot],
                dst_ref=comm_ref.at[recv_slot],
                send_sem=send_sems.at[s],
                recv_sem=recv_sems.at[s],
                device_id=(right,),
                device_id_type=pl.DeviceIdType.MESH,
            )
            rdma.start()
            rdma.wait()

            if s < N_DEV - 2:
                acc_into(recv_slot, (me - s - 1) % N_DEV)
            elif s == N_DEV - 2:
                c = (me + 1) % N_DEV
                acc_into(recv_slot, c)
                o_ref[pl.ds(c * CM, CM), :] = comm_ref[recv_slot]
            else:
                c = (me - (s - (N_DEV - 1))) % N_DEV
                o_ref[pl.ds(c * CM, CM), :] = comm_ref[recv_slot]

    return pl.pallas_call(
        body,
        out_shape=jax.ShapeDtypeStruct((M, N), jnp.bfloat16),
        in_specs=[pl.BlockSpec(memory_space=pl.ANY)],
        out_specs=pl.BlockSpec(memory_space=pltpu.VMEM),
        scratch_shapes=[
            pltpu.VMEM((2, CM, N), jnp.bfloat16),
            pltpu.VMEM((CM, N), jnp.bfloat16),
            pltpu.SemaphoreType.DMA((2 * (N_DEV - 1),)),
            pltpu.SemaphoreType.DMA((2 * (N_DEV - 1),)),
            pltpu.SemaphoreType.DMA,
        ],
        compiler_params=pltpu.CompilerParams(
            collective_id=0, vmem_limit_bytes=64 * 1024 * 1024
        ),
    )(P)


def kernel(A, B):
    P = _local_matmul(A, B)
    return _ring_allreduce(P)


# device time: 440804 ns/iter; 1.6360x vs baseline; 1.6360x over previous
import jax
import jax.numpy as jnp
from jax import lax
from jax.experimental import pallas as pl
from jax.experimental.pallas import tpu as pltpu

N_DEV = 4
N_HOPS = 2 * (N_DEV - 1)


def _local_matmul(A, B):
    M, K = A.shape
    _, N = B.shape
    BM = 512

    def mm_body(a_ref, b_ref, p_ref):
        p_ref[...] = jnp.dot(
            a_ref[...], b_ref[...], preferred_element_type=jnp.float32
        ).astype(jnp.bfloat16)

    return pl.pallas_call(
        mm_body,
        grid=(M // BM,),
        in_specs=[
            pl.BlockSpec((BM, K), lambda i: (i, 0)),
            pl.BlockSpec((K, N), lambda i: (0, 0)),
        ],
        out_specs=pl.BlockSpec((BM, N), lambda i: (i, 0)),
        out_shape=jax.ShapeDtypeStruct((M, N), jnp.bfloat16),
        compiler_params=pltpu.CompilerParams(
            vmem_limit_bytes=64 * 1024 * 1024
        ),
    )(A.astype(jnp.bfloat16), B.astype(jnp.bfloat16))


def _ring_allreduce(P):
    M, N = P.shape
    CM = M // N_DEV
    NH = N // 2

    def body(p_ref, o_ref, comm_p, comm_m, pch_p, pch_m,
             send_p, recv_p, send_m, recv_m, load_p, load_m):
        me = lax.axis_index("i")
        left = (me + N_DEV - 1) % N_DEV
        right = (me + 1) % N_DEV

        barrier_sem = pltpu.get_barrier_semaphore()
        for nbr in (left, right):
            pl.semaphore_signal(
                barrier_sem, inc=1,
                device_id=(nbr,), device_id_type=pl.DeviceIdType.MESH,
            )
        pl.semaphore_wait(barrier_sem, 2)

        def start_load(c, col0, dst, sem):
            cp = pltpu.make_async_copy(
                p_ref.at[pl.ds(c * CM, CM), pl.ds(col0, NH)], dst, sem
            )
            cp.start()
            return cp

        ld_p = start_load(me, 0, comm_p.at[0], load_p)
        ld_m = start_load(me, NH, comm_m.at[0], load_m)
        ld_p.wait()
        ld_m.wait()

        for s in range(N_HOPS):
            send_slot = s % 2
            recv_slot = (s + 1) % 2
            rdma_p = pltpu.make_async_remote_copy(
                src_ref=comm_p.at[send_slot],
                dst_ref=comm_p.at[recv_slot],
                send_sem=send_p.at[s],
                recv_sem=recv_p.at[s],
                device_id=(right,),
                device_id_type=pl.DeviceIdType.MESH,
            )
            rdma_m = pltpu.make_async_remote_copy(
                src_ref=comm_m.at[send_slot],
                dst_ref=comm_m.at[recv_slot],
                send_sem=send_m.at[s],
                recv_sem=recv_m.at[s],
                device_id=(left,),
                device_id_type=pl.DeviceIdType.MESH,
            )
            rdma_p.start()
            rdma_m.start()

            if s < N_DEV - 1:
                cp = (me - s - 1) % N_DEV
                cm = (me + s + 1) % N_DEV
                ld_p = start_load(cp, 0, pch_p, load_p)
                ld_m = start_load(cm, NH, pch_m, load_m)

            rdma_p.wait()
            rdma_m.wait()

            if s < N_DEV - 1:
                ld_p.wait()
                ld_m.wait()
                acc_p = (
                    comm_p[recv_slot].astype(jnp.float32)
                    + pch_p[...].astype(jnp.float32)
                ).astype(jnp.bfloat16)
                comm_p[recv_slot] = acc_p
                acc_m = (
                    comm_m[recv_slot].astype(jnp.float32)
                    + pch_m[...].astype(jnp.float32)
                ).astype(jnp.bfloat16)
                comm_m[recv_slot] = acc_m
                if s == N_DEV - 2:
                    o_ref[pl.ds(cp * CM, CM), pl.ds(0, NH)] = acc_p
                    o_ref[pl.ds(cm * CM, CM), pl.ds(NH, NH)] = acc_m
            else:
                t = s - (N_DEV - 1)
                gp = (me - t) % N_DEV
                gm = (me + t) % N_DEV
                o_ref[pl.ds(gp * CM, CM), pl.ds(0, NH)] = comm_p[recv_slot]
                o_ref[pl.ds(gm * CM, CM), pl.ds(NH, NH)] = comm_m[recv_slot]

    return pl.pallas_call(
        body,
        out_shape=jax.ShapeDtypeStruct((M, N), jnp.bfloat16),
        in_specs=[pl.BlockSpec(memory_space=pl.ANY)],
        out_specs=pl.BlockSpec(memory_space=pltpu.VMEM),
        scratch_shapes=[
            pltpu.VMEM((2, CM, NH), jnp.bfloat16),
            pltpu.VMEM((2, CM, NH), jnp.bfloat16),
            pltpu.VMEM((CM, NH), jnp.bfloat16),
            pltpu.VMEM((CM, NH), jnp.bfloat16),
            pltpu.SemaphoreType.DMA((N_HOPS,)),
            pltpu.SemaphoreType.DMA((N_HOPS,)),
            pltpu.SemaphoreType.DMA((N_HOPS,)),
            pltpu.SemaphoreType.DMA((N_HOPS,)),
            pltpu.SemaphoreType.DMA,
            pltpu.SemaphoreType.DMA,
        ],
        compiler_params=pltpu.CompilerParams(
            collective_id=0, vmem_limit_bytes=64 * 1024 * 1024
        ),
    )(P)


def kernel(A, B):
    P = _local_matmul(A, B)
    return _ring_allreduce(P)


# device time: 390232 ns/iter; 1.8480x vs baseline; 1.1296x over previous
import jax
import jax.numpy as jnp
from jax import lax
from jax.experimental import pallas as pl
from jax.experimental.pallas import tpu as pltpu

N_DEV = 4
N_HOPS = 2 * (N_DEV - 1)


def _fused_matmul_allreduce(A, B):
    M, K = A.shape
    _, N = B.shape
    CM = M // N_DEV
    NH = N // 2

    def body(a_ref, b_ref, o_ref, comm_p, comm_m, pch_p, pch_m,
             send_p, recv_p, send_m, recv_m, out_sem_p, out_sem_m):
        me = lax.axis_index("i")
        left = (me + N_DEV - 1) % N_DEV
        right = (me + 1) % N_DEV

        barrier_sem = pltpu.get_barrier_semaphore()
        for nbr in (left, right):
            pl.semaphore_signal(
                barrier_sem, inc=1,
                device_id=(nbr,), device_id_type=pl.DeviceIdType.MESH,
            )
        pl.semaphore_wait(barrier_sem, 2)

        def phalf(c, col0):
            a_rows = a_ref[pl.ds(c * CM, CM), :]
            b_cols = b_ref[:, pl.ds(col0, NH)]
            return jnp.dot(
                a_rows, b_cols, preferred_element_type=jnp.float32
            ).astype(jnp.bfloat16)

        def store_out(c, col0, src, sem):
            cp = pltpu.make_async_copy(
                src, o_ref.at[pl.ds(c * CM, CM), pl.ds(col0, NH)], sem
            )
            cp.start()
            return cp

        comm_p[0] = phalf(me, 0)
        comm_m[0] = phalf(me, NH)

        for s in range(N_HOPS):
            send_slot = s % 2
            recv_slot = (s + 1) % 2
            rdma_p = pltpu.make_async_remote_copy(
                src_ref=comm_p.at[send_slot],
                dst_ref=comm_p.at[recv_slot],
                send_sem=send_p.at[s],
                recv_sem=recv_p.at[s],
                device_id=(right,),
                device_id_type=pl.DeviceIdType.MESH,
            )
            rdma_m = pltpu.make_async_remote_copy(
                src_ref=comm_m.at[send_slot],
                dst_ref=comm_m.at[recv_slot],
                send_sem=send_m.at[s],
                recv_sem=recv_m.at[s],
                device_id=(left,),
                device_id_type=pl.DeviceIdType.MESH,
            )
            rdma_p.start()
            rdma_m.start()

            if s < N_DEV - 1:
                cp = (me - s - 1) % N_DEV
                cm = (me + s + 1) % N_DEV
                pch_p[...] = phalf(cp, 0)
                pch_m[...] = phalf(cm, NH)

            rdma_p.wait()
            rdma_m.wait()

            if s < N_DEV - 1:
                acc_p = (
                    comm_p[recv_slot].astype(jnp.float32)
                    + pch_p[...].astype(jnp.float32)
                ).astype(jnp.bfloat16)
                comm_p[recv_slot] = acc_p
                acc_m = (
                    comm_m[recv_slot].astype(jnp.float32)
                    + pch_m[...].astype(jnp.float32)
                ).astype(jnp.bfloat16)
                comm_m[recv_slot] = acc_m
                if s == N_DEV - 2:
                    st_p = store_out(cp, 0, comm_p.at[recv_slot], out_sem_p)
                    st_m = store_out(cm, NH, comm_m.at[recv_slot], out_sem_m)
                    st_p.wait()
                    st_m.wait()
            else:
                t = s - (N_DEV - 1)
                gp = (me - t) % N_DEV
                gm = (me + t) % N_DEV
                st_p = store_out(gp, 0, comm_p.at[recv_slot], out_sem_p)
                st_m = store_out(gm, NH, comm_m.at[recv_slot], out_sem_m)
                st_p.wait()
                st_m.wait()

    return pl.pallas_call(
        body,
        out_shape=jax.ShapeDtypeStruct((M, N), jnp.bfloat16),
        in_specs=[
            pl.BlockSpec(memory_space=pltpu.VMEM),
            pl.BlockSpec(memory_space=pltpu.VMEM),
        ],
        out_specs=pl.BlockSpec(memory_space=pl.ANY),
        scratch_shapes=[
            pltpu.VMEM((2, CM, NH), jnp.bfloat16),
            pltpu.VMEM((2, CM, NH), jnp.bfloat16),
            pltpu.VMEM((CM, NH), jnp.bfloat16),
            pltpu.VMEM((CM, NH), jnp.bfloat16),
            pltpu.SemaphoreType.DMA((N_HOPS,)),
            pltpu.SemaphoreType.DMA((N_HOPS,)),
            pltpu.SemaphoreType.DMA((N_HOPS,)),
            pltpu.SemaphoreType.DMA((N_HOPS,)),
            pltpu.SemaphoreType.DMA,
            pltpu.SemaphoreType.DMA,
        ],
        compiler_params=pltpu.CompilerParams(
            collective_id=0, vmem_limit_bytes=64 * 1024 * 1024
        ),
    )(A.astype(jnp.bfloat16), B.astype(jnp.bfloat16))


def kernel(A, B):
    return _fused_matmul_allreduce(A, B)


# device time: 380549 ns/iter; 1.8950x vs baseline; 1.0254x over previous
import jax
import jax.numpy as jnp
from jax import lax
from jax.experimental import pallas as pl
from jax.experimental.pallas import tpu as pltpu

N_DEV = 4
N_HOPS = 2 * (N_DEV - 1)


def _fused_matmul_allreduce(A, B):
    M, K = A.shape
    _, N = B.shape
    CM = M // N_DEV
    NH = N // 2

    def body(a_ref, b_ref, o_ref, comm_p, comm_m, pch_p, pch_m,
             send_p, recv_p, send_m, recv_m, out_sems_p, out_sems_m):
        me = lax.axis_index("i")
        left = (me + N_DEV - 1) % N_DEV
        right = (me + 1) % N_DEV

        barrier_sem = pltpu.get_barrier_semaphore()
        for nbr in (left, right):
            pl.semaphore_signal(
                barrier_sem, inc=1,
                device_id=(nbr,), device_id_type=pl.DeviceIdType.MESH,
            )
        pl.semaphore_wait(barrier_sem, 2)

        def phalf(c, col0):
            a_rows = a_ref[pl.ds(c * CM, CM), :]
            b_cols = b_ref[:, pl.ds(col0, NH)]
            return jnp.dot(
                a_rows, b_cols, preferred_element_type=jnp.float32
            ).astype(jnp.bfloat16)

        def store_out(c, col0, src, sem):
            cp = pltpu.make_async_copy(
                src, o_ref.at[pl.ds(c * CM, CM), pl.ds(col0, NH)], sem
            )
            cp.start()
            return cp

        def mk_rdma(comm, sends, recvs, s, target):
            return pltpu.make_async_remote_copy(
                src_ref=comm.at[s % 2],
                dst_ref=comm.at[(s + 1) % 2],
                send_sem=sends.at[s],
                recv_sem=recvs.at[s],
                device_id=(target,),
                device_id_type=pl.DeviceIdType.MESH,
            )

        comm_p[0] = phalf(me, 0)
        mk_rdma(comm_p, send_p, recv_p, 0, right).start()
        comm_m[0] = phalf(me, NH)
        mk_rdma(comm_m, send_m, recv_m, 0, left).start()

        out_stores = []

        for s in range(N_HOPS):
            recv_slot = (s + 1) % 2
            rdma_p = mk_rdma(comm_p, send_p, recv_p, s, right)
            rdma_m = mk_rdma(comm_m, send_m, recv_m, s, left)
            if s > 0:
                rdma_p.start()
                rdma_m.start()

            if s < N_DEV - 1:
                cp = (me - s - 1) % N_DEV
                cm = (me + s + 1) % N_DEV
                pch_p[...] = phalf(cp, 0)
                pch_m[...] = phalf(cm, NH)

            rdma_p.wait()
            rdma_m.wait()

            if s < N_DEV - 1:
                acc_p = (
                    comm_p[recv_slot].astype(jnp.float32)
                    + pch_p[...].astype(jnp.float32)
                ).astype(jnp.bfloat16)
                comm_p[recv_slot] = acc_p
                acc_m = (
                    comm_m[recv_slot].astype(jnp.float32)
                    + pch_m[...].astype(jnp.float32)
                ).astype(jnp.bfloat16)
                comm_m[recv_slot] = acc_m
                if s == N_DEV - 2:
                    out_stores.append(
                        store_out(cp, 0, comm_p.at[recv_slot], out_sems_p.at[s - 2])
                    )
                    out_stores.append(
                        store_out(cm, NH, comm_m.at[recv_slot], out_sems_m.at[s - 2])
                    )
            else:
                t = s - (N_DEV - 1)
                gp = (me - t) % N_DEV
                gm = (me + t) % N_DEV
                out_stores.append(
                    store_out(gp, 0, comm_p.at[recv_slot], out_sems_p.at[s - 2])
                )
                out_stores.append(
                    store_out(gm, NH, comm_m.at[recv_slot], out_sems_m.at[s - 2])
                )

        for st in out_stores:
            st.wait()

    return pl.pallas_call(
        body,
        out_shape=jax.ShapeDtypeStruct((M, N), jnp.bfloat16),
        in_specs=[
            pl.BlockSpec(memory_space=pltpu.VMEM),
            pl.BlockSpec(memory_space=pltpu.VMEM),
        ],
        out_specs=pl.BlockSpec(memory_space=pl.ANY),
        scratch_shapes=[
            pltpu.VMEM((2, CM, NH), jnp.bfloat16),
            pltpu.VMEM((2, CM, NH), jnp.bfloat16),
            pltpu.VMEM((CM, NH), jnp.bfloat16),
            pltpu.VMEM((CM, NH), jnp.bfloat16),
            pltpu.SemaphoreType.DMA((N_HOPS,)),
            pltpu.SemaphoreType.DMA((N_HOPS,)),
            pltpu.SemaphoreType.DMA((N_HOPS,)),
            pltpu.SemaphoreType.DMA((N_HOPS,)),
            pltpu.SemaphoreType.DMA((N_DEV,)),
            pltpu.SemaphoreType.DMA((N_DEV,)),
        ],
        compiler_params=pltpu.CompilerParams(
            collective_id=0, vmem_limit_bytes=64 * 1024 * 1024
        ),
    )(A.astype(jnp.bfloat16), B.astype(jnp.bfloat16))


def kernel(A, B):
    return _fused_matmul_allreduce(A, B)


# device time: 356625 ns/iter; 2.0221x vs baseline; 1.0671x over previous
import jax
import jax.numpy as jnp
from jax import lax
from jax.experimental import pallas as pl
from jax.experimental.pallas import tpu as pltpu

N_DEV = 4
N_HOPS = 2 * (N_DEV - 1)


def _fused_matmul_allreduce(A, B):
    M, K = A.shape
    _, N = B.shape
    CM = M // N_DEV
    NH = N // 2

    def body(a_ref, b_ref, o_ref, comm_p, comm_m, pch_p, pch_m, a_stage, a16,
             send_p, recv_p, send_m, recv_m, out_sems_p, out_sems_m, a_sem):
        me = lax.axis_index("i")
        left = (me + N_DEV - 1) % N_DEV
        right = (me + 1) % N_DEV

        barrier_sem = pltpu.get_barrier_semaphore()
        for nbr in (left, right):
            pl.semaphore_signal(
                barrier_sem, inc=1,
                device_id=(nbr,), device_id_type=pl.DeviceIdType.MESH,
            )
        pl.semaphore_wait(barrier_sem, 2)

        def load_a(c):
            cp = pltpu.make_async_copy(
                a_ref.at[pl.ds(c * CM, CM), :], a_stage, a_sem
            )
            cp.start()
            cp.wait()
            a16[...] = a_stage[...].astype(jnp.bfloat16)

        def phalf(col0):
            return jnp.dot(
                a16[...], b_ref[:, pl.ds(col0, NH)],
                preferred_element_type=jnp.float32,
            ).astype(jnp.bfloat16)

        def store_out(c, col0, src, sem):
            cp = pltpu.make_async_copy(
                src, o_ref.at[pl.ds(c * CM, CM), pl.ds(col0, NH)], sem
            )
            cp.start()
            return cp

        def mk_rdma(comm, sends, recvs, s, target):
            return pltpu.make_async_remote_copy(
                src_ref=comm.at[s % 2],
                dst_ref=comm.at[(s + 1) % 2],
                send_sem=sends.at[s],
                recv_sem=recvs.at[s],
                device_id=(target,),
                device_id_type=pl.DeviceIdType.MESH,
            )

        load_a(me)
        comm_p[0] = phalf(0)
        mk_rdma(comm_p, send_p, recv_p, 0, right).start()
        comm_m[0] = phalf(NH)
        mk_rdma(comm_m, send_m, recv_m, 0, left).start()

        out_stores = []

        for s in range(N_HOPS):
            recv_slot = (s + 1) % 2
            rdma_p = mk_rdma(comm_p, send_p, recv_p, s, right)
            rdma_m = mk_rdma(comm_m, send_m, recv_m, s, left)
            if s > 0:
                rdma_p.start()
                rdma_m.start()

            if s < N_DEV - 1:
                cp = (me - s - 1) % N_DEV
                cm = (me + s + 1) % N_DEV
                load_a(cp)
                pch_p[...] = phalf(0)
                load_a(cm)
                pch_m[...] = phalf(NH)

            rdma_p.wait()
            rdma_m.wait()

            if s < N_DEV - 1:
                comm_p[recv_slot] = comm_p[recv_slot] + pch_p[...]
                comm_m[recv_slot] = comm_m[recv_slot] + pch_m[...]
                if s == N_DEV - 2:
                    out_stores.append(
                        store_out(cp, 0, comm_p.at[recv_slot], out_sems_p.at[s - 2])
                    )
                    out_stores.append(
                        store_out(cm, NH, comm_m.at[recv_slot], out_sems_m.at[s - 2])
                    )
            else:
                t = s - (N_DEV - 1)
                gp = (me - t) % N_DEV
                gm = (me + t) % N_DEV
                out_stores.append(
                    store_out(gp, 0, comm_p.at[recv_slot], out_sems_p.at[s - 2])
                )
                out_stores.append(
                    store_out(gm, NH, comm_m.at[recv_slot], out_sems_m.at[s - 2])
                )

        for st in out_stores:
            st.wait()

    return pl.pallas_call(
        body,
        out_shape=jax.ShapeDtypeStruct((M, N), jnp.bfloat16),
        in_specs=[
            pl.BlockSpec(memory_space=pl.ANY),
            pl.BlockSpec(memory_space=pltpu.VMEM),
        ],
        out_specs=pl.BlockSpec(memory_space=pl.ANY),
        scratch_shapes=[
            pltpu.VMEM((2, CM, NH), jnp.bfloat16),
            pltpu.VMEM((2, CM, NH), jnp.bfloat16),
            pltpu.VMEM((CM, NH), jnp.bfloat16),
            pltpu.VMEM((CM, NH), jnp.bfloat16),
            pltpu.VMEM((CM, K), jnp.float32),
            pltpu.VMEM((CM, K), jnp.bfloat16),
            pltpu.SemaphoreType.DMA((N_HOPS,)),
            pltpu.SemaphoreType.DMA((N_HOPS,)),
            pltpu.SemaphoreType.DMA((N_HOPS,)),
            pltpu.SemaphoreType.DMA((N_HOPS,)),
            pltpu.SemaphoreType.DMA((N_DEV,)),
            pltpu.SemaphoreType.DMA((N_DEV,)),
            pltpu.SemaphoreType.DMA,
        ],
        compiler_params=pltpu.CompilerParams(
            collective_id=0, vmem_limit_bytes=64 * 1024 * 1024
        ),
    )(A, B.astype(jnp.bfloat16))


def kernel(A, B):
    return _fused_matmul_allreduce(A, B)


# device time: 336426 ns/iter; 2.1436x vs baseline; 1.0600x over previous
import jax
import jax.numpy as jnp
from jax import lax
from jax.experimental import pallas as pl
from jax.experimental.pallas import tpu as pltpu

N_DEV = 4
N_HOPS = 2 * (N_DEV - 1)
N_SUB = 2


def _fused_matmul_allreduce(A, B):
    M, K = A.shape
    _, N = B.shape
    CM = M // N_DEV
    NH = N // 2
    HM = CM // N_SUB

    def body(a_ref, b_ref, o_ref, comm_p, comm_m, pch_p, pch_m, a_stage, a16,
             send_p, recv_p, send_m, recv_m, out_sems_p, out_sems_m, a_sem):
        me = lax.axis_index("i")
        left = (me + N_DEV - 1) % N_DEV
        right = (me + 1) % N_DEV

        barrier_sem = pltpu.get_barrier_semaphore()
        for nbr in (left, right):
            pl.semaphore_signal(
                barrier_sem, inc=1,
                device_id=(nbr,), device_id_type=pl.DeviceIdType.MESH,
            )
        pl.semaphore_wait(barrier_sem, 2)

        def load_a(c):
            cp = pltpu.make_async_copy(
                a_ref.at[pl.ds(c * CM, CM), :], a_stage, a_sem
            )
            cp.start()
            cp.wait()
            a16[...] = a_stage[...].astype(jnp.bfloat16)

        def phalf_sub(col0, k):
            return jnp.dot(
                a16[pl.ds(k * HM, HM), :], b_ref[:, pl.ds(col0, NH)],
                preferred_element_type=jnp.float32,
            ).astype(jnp.bfloat16)

        def store_out(c, col0, src, sem):
            cp = pltpu.make_async_copy(
                src, o_ref.at[pl.ds(c * CM, CM), pl.ds(col0, NH)], sem
            )
            cp.start()
            return cp

        def mk_rdma(comm, sends, recvs, s, k, target):
            return pltpu.make_async_remote_copy(
                src_ref=comm.at[s % 2, pl.ds(k * HM, HM), :],
                dst_ref=comm.at[(s + 1) % 2, pl.ds(k * HM, HM), :],
                send_sem=sends.at[s, k],
                recv_sem=recvs.at[s, k],
                device_id=(target,),
                device_id_type=pl.DeviceIdType.MESH,
            )

        load_a(me)
        for k in range(N_SUB):
            comm_p[0, pl.ds(k * HM, HM), :] = phalf_sub(0, k)
            mk_rdma(comm_p, send_p, recv_p, 0, k, right).start()
            comm_m[0, pl.ds(k * HM, HM), :] = phalf_sub(NH, k)
            mk_rdma(comm_m, send_m, recv_m, 0, k, left).start()

        out_stores = []

        for s in range(N_HOPS):
            recv_slot = (s + 1) % 2
            if s < N_DEV - 1:
                cp = (me - s - 1) % N_DEV
                cm = (me + s + 1) % N_DEV
                load_a(cp)
                pch_p[...] = jnp.dot(
                    a16[...], b_ref[:, pl.ds(0, NH)],
                    preferred_element_type=jnp.float32,
                ).astype(jnp.bfloat16)
                load_a(cm)
                pch_m[...] = jnp.dot(
                    a16[...], b_ref[:, pl.ds(NH, NH)],
                    preferred_element_type=jnp.float32,
                ).astype(jnp.bfloat16)

            for k in range(N_SUB):
                rows = pl.ds(k * HM, HM)
                mk_rdma(comm_p, send_p, recv_p, s, k, right).wait()
                if s < N_DEV - 1:
                    comm_p[recv_slot, rows, :] = (
                        comm_p[recv_slot, rows, :] + pch_p[rows, :]
                    )
                if s + 1 < N_HOPS:
                    mk_rdma(comm_p, send_p, recv_p, s + 1, k, right).start()

                mk_rdma(comm_m, send_m, recv_m, s, k, left).wait()
                if s < N_DEV - 1:
                    comm_m[recv_slot, rows, :] = (
                        comm_m[recv_slot, rows, :] + pch_m[rows, :]
                    )
                if s + 1 < N_HOPS:
                    mk_rdma(comm_m, send_m, recv_m, s + 1, k, left).start()

            if s == N_DEV - 2:
                out_stores.append(
                    store_out(cp, 0, comm_p.at[recv_slot], out_sems_p.at[s - 2])
                )
                out_stores.append(
                    store_out(cm, NH, comm_m.at[recv_slot], out_sems_m.at[s - 2])
                )
            elif s > N_DEV - 2:
                t = s - (N_DEV - 1)
                gp = (me - t) % N_DEV
                gm = (me + t) % N_DEV
                out_stores.append(
                    store_out(gp, 0, comm_p.at[recv_slot], out_sems_p.at[s - 2])
                )
                out_stores.append(
                    store_out(gm, NH, comm_m.at[recv_slot], out_sems_m.at[s - 2])
                )

        for st in out_stores:
            st.wait()

    return pl.pallas_call(
        body,
        out_shape=jax.ShapeDtypeStruct((M, N), jnp.bfloat16),
        in_specs=[
            pl.BlockSpec(memory_space=pl.ANY),
            pl.BlockSpec(memory_space=pltpu.VMEM),
        ],
        out_specs=pl.BlockSpec(memory_space=pl.ANY),
        scratch_shapes=[
            pltpu.VMEM((2, CM, NH), jnp.bfloat16),
            pltpu.VMEM((2, CM, NH), jnp.bfloat16),
            pltpu.VMEM((CM, NH), jnp.bfloat16),
            pltpu.VMEM((CM, NH), jnp.bfloat16),
            pltpu.VMEM((CM, K), jnp.float32),
            pltpu.VMEM((CM, K), jnp.bfloat16),
            pltpu.SemaphoreType.DMA((N_HOPS, N_SUB)),
            pltpu.SemaphoreType.DMA((N_HOPS, N_SUB)),
            pltpu.SemaphoreType.DMA((N_HOPS, N_SUB)),
            pltpu.SemaphoreType.DMA((N_HOPS, N_SUB)),
            pltpu.SemaphoreType.DMA((N_DEV,)),
            pltpu.SemaphoreType.DMA((N_DEV,)),
            pltpu.SemaphoreType.DMA,
        ],
        compiler_params=pltpu.CompilerParams(
            collective_id=0, vmem_limit_bytes=64 * 1024 * 1024
        ),
    )(A, B.astype(jnp.bfloat16))


def kernel(A, B):
    return _fused_matmul_allreduce(A, B)


# device time: 332580 ns/iter; 2.1683x vs baseline; 1.0116x over previous
import jax
import jax.numpy as jnp
from jax import lax
from jax.experimental import pallas as pl
from jax.experimental.pallas import tpu as pltpu

N_DEV = 4
N_HOPS = 2 * (N_DEV - 1)
N_SUB = 4


def _fused_matmul_allreduce(A, B):
    M, K = A.shape
    _, N = B.shape
    CM = M // N_DEV
    NH = N // 2
    HM = CM // N_SUB

    def body(a_ref, b_ref, o_ref, comm_p, comm_m, pch_p, pch_m, a_stage, a16,
             send_p, recv_p, send_m, recv_m, out_sems_p, out_sems_m, a_sem):
        me = lax.axis_index("i")
        left = (me + N_DEV - 1) % N_DEV
        right = (me + 1) % N_DEV

        barrier_sem = pltpu.get_barrier_semaphore()
        for nbr in (left, right):
            pl.semaphore_signal(
                barrier_sem, inc=1,
                device_id=(nbr,), device_id_type=pl.DeviceIdType.MESH,
            )
        pl.semaphore_wait(barrier_sem, 2)

        def load_a(c):
            cp = pltpu.make_async_copy(
                a_ref.at[pl.ds(c * CM, CM), :], a_stage, a_sem
            )
            cp.start()
            cp.wait()
            a16[...] = a_stage[...].astype(jnp.bfloat16)

        def phalf_sub(col0, k):
            return jnp.dot(
                a16[pl.ds(k * HM, HM), :], b_ref[:, pl.ds(col0, NH)],
                preferred_element_type=jnp.float32,
            ).astype(jnp.bfloat16)

        def store_out(c, col0, src, sem):
            cp = pltpu.make_async_copy(
                src, o_ref.at[pl.ds(c * CM, CM), pl.ds(col0, NH)], sem
            )
            cp.start()
            return cp

        def mk_rdma(comm, sends, recvs, s, k, target):
            return pltpu.make_async_remote_copy(
                src_ref=comm.at[s % 2, pl.ds(k * HM, HM), :],
                dst_ref=comm.at[(s + 1) % 2, pl.ds(k * HM, HM), :],
                send_sem=sends.at[s, k],
                recv_sem=recvs.at[s, k],
                device_id=(target,),
                device_id_type=pl.DeviceIdType.MESH,
            )

        load_a(me)
        for k in range(N_SUB):
            comm_p[0, pl.ds(k * HM, HM), :] = phalf_sub(0, k)
            mk_rdma(comm_p, send_p, recv_p, 0, k, right).start()
            comm_m[0, pl.ds(k * HM, HM), :] = phalf_sub(NH, k)
            mk_rdma(comm_m, send_m, recv_m, 0, k, left).start()

        out_stores = []

        for s in range(N_HOPS):
            recv_slot = (s + 1) % 2
            if s < N_DEV - 1:
                cp = (me - s - 1) % N_DEV
                cm = (me + s + 1) % N_DEV
                load_a(cp)
                pch_p[...] = jnp.dot(
                    a16[...], b_ref[:, pl.ds(0, NH)],
                    preferred_element_type=jnp.float32,
                ).astype(jnp.bfloat16)
                load_a(cm)
                pch_m[...] = jnp.dot(
                    a16[...], b_ref[:, pl.ds(NH, NH)],
                    preferred_element_type=jnp.float32,
                ).astype(jnp.bfloat16)

            for k in range(N_SUB):
                rows = pl.ds(k * HM, HM)
                mk_rdma(comm_p, send_p, recv_p, s, k, right).wait()
                if s < N_DEV - 1:
                    comm_p[recv_slot, rows, :] = (
                        comm_p[recv_slot, rows, :] + pch_p[rows, :]
                    )
                if s + 1 < N_HOPS:
                    mk_rdma(comm_p, send_p, recv_p, s + 1, k, right).start()

                mk_rdma(comm_m, send_m, recv_m, s, k, left).wait()
                if s < N_DEV - 1:
                    comm_m[recv_slot, rows, :] = (
                        comm_m[recv_slot, rows, :] + pch_m[rows, :]
                    )
                if s + 1 < N_HOPS:
                    mk_rdma(comm_m, send_m, recv_m, s + 1, k, left).start()

            if s == N_DEV - 2:
                out_stores.append(
                    store_out(cp, 0, comm_p.at[recv_slot], out_sems_p.at[s - 2])
                )
                out_stores.append(
                    store_out(cm, NH, comm_m.at[recv_slot], out_sems_m.at[s - 2])
                )
            elif s > N_DEV - 2:
                t = s - (N_DEV - 1)
                gp = (me - t) % N_DEV
                gm = (me + t) % N_DEV
                out_stores.append(
                    store_out(gp, 0, comm_p.at[recv_slot], out_sems_p.at[s - 2])
                )
                out_stores.append(
                    store_out(gm, NH, comm_m.at[recv_slot], out_sems_m.at[s - 2])
                )

        for st in out_stores:
            st.wait()

    return pl.pallas_call(
        body,
        out_shape=jax.ShapeDtypeStruct((M, N), jnp.bfloat16),
        in_specs=[
            pl.BlockSpec(memory_space=pl.ANY),
            pl.BlockSpec(memory_space=pltpu.VMEM),
        ],
        out_specs=pl.BlockSpec(memory_space=pl.ANY),
        scratch_shapes=[
            pltpu.VMEM((2, CM, NH), jnp.bfloat16),
            pltpu.VMEM((2, CM, NH), jnp.bfloat16),
            pltpu.VMEM((CM, NH), jnp.bfloat16),
            pltpu.VMEM((CM, NH), jnp.bfloat16),
            pltpu.VMEM((CM, K), jnp.float32),
            pltpu.VMEM((CM, K), jnp.bfloat16),
            pltpu.SemaphoreType.DMA((N_HOPS, N_SUB)),
            pltpu.SemaphoreType.DMA((N_HOPS, N_SUB)),
            pltpu.SemaphoreType.DMA((N_HOPS, N_SUB)),
            pltpu.SemaphoreType.DMA((N_HOPS, N_SUB)),
            pltpu.SemaphoreType.DMA((N_DEV,)),
            pltpu.SemaphoreType.DMA((N_DEV,)),
            pltpu.SemaphoreType.DMA,
        ],
        compiler_params=pltpu.CompilerParams(
            collective_id=0, vmem_limit_bytes=64 * 1024 * 1024
        ),
    )(A, B.astype(jnp.bfloat16))


def kernel(A, B):
    return _fused_matmul_allreduce(A, B)
